# Initial kernel scaffold; baseline (speedup 1.0000x reference)
import functools

import jax
import jax.numpy as jnp
from jax import lax
from jax.experimental import pallas as pl
from jax.experimental.pallas import tpu as pltpu
from jax.experimental.pallas import tpu_sc as plsc

B = 16384
F = 26
K = 32
N_WORKERS = 32
CHUNK = B // N_WORKERS
SUB = 32
NSUB = CHUNK // SUB
L = 16

_mesh = plsc.VectorSubcoreMesh(core_axis_name="c", subcore_axis_name="s")


@functools.partial(
    pl.kernel,
    mesh=_mesh,
    out_type=jax.ShapeDtypeStruct((B,), jnp.float32),
    scratch_types=[
        pltpu.VMEM((CHUNK * F,), jnp.int32),
        pltpu.VMEM((CHUNK * F,), jnp.float32),
        pltpu.VMEM((SUB * F, K), jnp.float32),
        pltpu.VMEM((SUB,), jnp.float32),
        pltpu.VMEM((L,), jnp.float32),
        pltpu.SemaphoreType.DMA,
        pltpu.SemaphoreType.DMA,
    ],
)
def _fm_sc(x_hbm, lw_hbm, vw_hbm, b_hbm, out_hbm,
           idx_v, lw_v, rows_v, out_v, bias_v, sem_rows, sem_lw):
    wid = lax.axis_index("s") * 2 + lax.axis_index("c")
    base = wid * CHUNK

    pltpu.sync_copy(x_hbm.at[pl.ds(base * F, CHUNK * F)], idx_v)
    pltpu.sync_copy(b_hbm, bias_v)
    bias_vec = bias_v[...]
    lane = lax.iota(jnp.int32, (L,), 0)

    pltpu.async_copy(lw_hbm.at[idx_v], lw_v, sem_lw).wait()

    def sub_body(j, _):
        sbase = j * SUB
        pltpu.async_copy(
            vw_hbm.at[idx_v.at[pl.ds(sbase * F, SUB * F)]], rows_v, sem_rows
        ).wait()

        for g in range(SUB // L):
            def s_body(s16, out_vec):
                s = g * L + s16
                rbase = s * F
                lbase = (sbase + s) * F
                a0 = jnp.zeros((L,), jnp.float32)
                a1 = jnp.zeros((L,), jnp.float32)
                sq = jnp.zeros((L,), jnp.float32)
                for f in range(F):
                    r0 = rows_v[rbase + f, pl.ds(0, L)]
                    r1 = rows_v[rbase + f, pl.ds(L, L)]
                    a0 = a0 + r0
                    a1 = a1 + r1
                    sq = sq + r0 * r0
                    sq = sq + r1 * r1
                l0 = lw_v[pl.ds(lbase, L)]
                l1 = lw_v[pl.ds(lbase + L, L)]
                linv = l0 + jnp.where(lane < (F - L), l1, 0.0)
                tot = 0.5 * (a0 * a0 + a1 * a1 - sq) + linv
                val = jnp.sum(tot)
                return jnp.where(lane == s16, val, out_vec)

            out_vec = lax.fori_loop(0, L, s_body, jnp.zeros((L,), jnp.float32))
            out_v[pl.ds(g * L, L)] = out_vec + bias_vec

        pltpu.sync_copy(out_v, out_hbm.at[pl.ds(base + sbase, SUB)])
        return 0

    lax.fori_loop(0, NSUB, sub_body, 0)


def kernel(x, linear_w, v_w, b):
    xf = x.reshape(-1).astype(jnp.int32)
    lwf = linear_w.reshape(-1)
    b16 = jnp.broadcast_to(b.astype(jnp.float32), (L,))
    out = _fm_sc(xf, lwf, v_w, b16)
    return out.reshape(B, 1)


# speedup vs baseline: 2.1546x; 2.1546x over previous
import functools

import jax
import jax.numpy as jnp
from jax import lax
from jax.experimental import pallas as pl
from jax.experimental.pallas import tpu as pltpu
from jax.experimental.pallas import tpu_sc as plsc

B = 16384
F = 26
K = 32
N_WORKERS = 32
CHUNK = B // N_WORKERS
SUB = 32
NSUB = CHUNK // SUB
L = 16

_mesh = plsc.VectorSubcoreMesh(core_axis_name="c", subcore_axis_name="s")

_GDN = lax.GatherDimensionNumbers(
    offset_dims=(), collapsed_slice_dims=(0,), start_index_map=(0,))


def _perm(x, idx):
    return lax.gather(x, idx[:, None], _GDN, slice_sizes=(1,),
                      mode=lax.GatherScatterMode.PROMISE_IN_BOUNDS)


def _lane_sum(x, lane):
    for d in (1, 2, 4, 8):
        x = x + _perm(x, lane ^ d)
    return x


@functools.partial(
    pl.kernel,
    mesh=_mesh,
    compiler_params=pltpu.CompilerParams(use_tc_tiling_on_sc=False),
    out_type=jax.ShapeDtypeStruct((B,), jnp.float32),
    scratch_types=[
        pltpu.VMEM((CHUNK * F,), jnp.int32),
        pltpu.VMEM((CHUNK * F,), jnp.float32),
        pltpu.VMEM((SUB * F, K), jnp.float32),
        pltpu.VMEM((SUB,), jnp.float32),
        pltpu.VMEM((L,), jnp.float32),
        pltpu.SemaphoreType.DMA,
        pltpu.SemaphoreType.DMA,
    ],
)
def _fm_sc(x_hbm, lw_hbm, vw_hbm, b_hbm, out_hbm,
           idx_v, lw_v, rows_v, out_v, bias_v, sem_rows, sem_lw):
    wid = lax.axis_index("s") * 2 + lax.axis_index("c")
    base = wid * CHUNK

    pltpu.sync_copy(x_hbm.at[pl.ds(base * F, CHUNK * F)], idx_v)
    pltpu.sync_copy(b_hbm, bias_v)
    bias_vec = bias_v[...]
    lane = lax.broadcasted_iota(jnp.int32, (L,), 0)

    pltpu.async_copy(lw_hbm.at[idx_v], lw_v, sem_lw).wait()

    def sub_body(j, _):
        sbase = j * SUB
        pltpu.async_copy(
            vw_hbm.at[idx_v.at[pl.ds(sbase * F, SUB * F)]], rows_v, sem_rows
        ).wait()

        for g in range(SUB // L):
            def s_body(s16, out_vec):
                s = g * L + s16
                rbase = s * F
                lbase = (sbase + s) * F
                a0 = jnp.zeros((L,), jnp.float32)
                a1 = jnp.zeros((L,), jnp.float32)
                sq = jnp.zeros((L,), jnp.float32)
                for f in range(F):
                    r0 = rows_v[rbase + f, pl.ds(0, L)]
                    r1 = rows_v[rbase + f, pl.ds(L, L)]
                    a0 = a0 + r0
                    a1 = a1 + r1
                    sq = sq + r0 * r0
                    sq = sq + r1 * r1
                l0 = lw_v[pl.ds(lbase, L)]
                l1 = lw_v[pl.ds(lbase + L, L)]
                linv = l0 + jnp.where(lane < (F - L), l1, 0.0)
                tot = 0.5 * (a0 * a0 + a1 * a1 - sq) + linv
                tot = _lane_sum(tot, lane)
                return jnp.where(lane == s16, tot, out_vec)

            out_vec = lax.fori_loop(0, L, s_body, jnp.zeros((L,), jnp.float32))
            out_v[pl.ds(g * L, L)] = out_vec + bias_vec

        pltpu.sync_copy(out_v, out_hbm.at[pl.ds(base + sbase, SUB)])
        return 0

    lax.fori_loop(0, NSUB, sub_body, 0)


def kernel(x, linear_w, v_w, b):
    xf = x.reshape(-1).astype(jnp.int32)
    lwf = linear_w.reshape(-1)
    b16 = jnp.broadcast_to(b.astype(jnp.float32), (L,))
    out = _fm_sc(xf, lwf, v_w, b16)
    return out.reshape(B, 1)


# speedup vs baseline: 2.2407x; 1.0400x over previous
import functools

import jax
import jax.numpy as jnp
from jax import lax
from jax.experimental import pallas as pl
from jax.experimental.pallas import tpu as pltpu
from jax.experimental.pallas import tpu_sc as plsc

B = 16384
F = 26
K = 32
N_WORKERS = 32
CHUNK = B // N_WORKERS
SUB = 32
NSUB = CHUNK // SUB
L = 16

_mesh = plsc.VectorSubcoreMesh(core_axis_name="c", subcore_axis_name="s")

_GDN = lax.GatherDimensionNumbers(
    offset_dims=(), collapsed_slice_dims=(0,), start_index_map=(0,))


def _perm(x, idx):
    return lax.gather(x, idx[:, None], _GDN, slice_sizes=(1,),
                      mode=lax.GatherScatterMode.PROMISE_IN_BOUNDS)


def _lane_sum(x, lane):
    for d in (1, 2, 4, 8):
        x = x + _perm(x, lane ^ d)
    return x


@functools.partial(
    pl.kernel,
    mesh=_mesh,
    compiler_params=pltpu.CompilerParams(use_tc_tiling_on_sc=False),
    out_type=jax.ShapeDtypeStruct((B,), jnp.float32),
    scratch_types=[
        pltpu.VMEM((CHUNK * F,), jnp.int32),
        pltpu.VMEM((CHUNK * F,), jnp.float32),
        pltpu.VMEM((2, SUB * F, K), jnp.float32),
        pltpu.VMEM((SUB,), jnp.float32),
        pltpu.VMEM((L,), jnp.float32),
        pltpu.SemaphoreType.DMA,
        pltpu.SemaphoreType.DMA,
        pltpu.SemaphoreType.DMA,
    ],
)
def _fm_sc(x_hbm, lw_hbm, vw_hbm, b_hbm, out_hbm,
           idx_v, lw_v, rows_v, out_v, bias_v, sem0, sem1, sem_lw):
    wid = lax.axis_index("s") * 2 + lax.axis_index("c")
    base = wid * CHUNK
    sems = (sem0, sem1)

    pltpu.sync_copy(x_hbm.at[pl.ds(base * F, CHUNK * F)], idx_v)
    pltpu.sync_copy(b_hbm, bias_v)
    bias_vec = bias_v[...]
    lane = lax.broadcasted_iota(jnp.int32, (L,), 0)

    def gather_rows(j, buf):
        return pltpu.make_async_copy(
            vw_hbm.at[idx_v.at[pl.ds(j * (SUB * F), SUB * F)]],
            rows_v.at[buf],
            sems[buf],
        )

    lw_copy = pltpu.make_async_copy(lw_hbm.at[idx_v], lw_v, sem_lw)
    lw_copy.start()
    gather_rows(0, 0).start()
    lw_copy.wait()

    def compute_sub(j, buf):
        rows = rows_v.at[buf]
        for g in range(SUB // L):
            def s_body(s16, out_vec):
                s = g * L + s16
                rbase = s * F
                lbase = j * (SUB * F) + s * F
                a0 = [None] * 4
                a1 = [None] * 4
                sq = [None] * 8
                for f in range(F):
                    r0 = rows[rbase + f, pl.ds(0, L)]
                    r1 = rows[rbase + f, pl.ds(L, L)]
                    p = f % 4
                    a0[p] = r0 if a0[p] is None else a0[p] + r0
                    a1[p] = r1 if a1[p] is None else a1[p] + r1
                    m0 = r0 * r0
                    m1 = r1 * r1
                    sq[p] = m0 if sq[p] is None else sq[p] + m0
                    sq[p + 4] = m1 if sq[p + 4] is None else sq[p + 4] + m1
                s0 = (a0[0] + a0[1]) + (a0[2] + a0[3])
                s1 = (a1[0] + a1[1]) + (a1[2] + a1[3])
                qq = ((sq[0] + sq[1]) + (sq[2] + sq[3])) + (
                    (sq[4] + sq[5]) + (sq[6] + sq[7]))
                l0 = lw_v[pl.ds(lbase, L)]
                l1 = lw_v[pl.ds(lbase + L, L)]
                linv = l0 + jnp.where(lane < (F - L), l1, 0.0)
                tot = 0.5 * (s0 * s0 + s1 * s1 - qq) + linv
                tot = _lane_sum(tot, lane)
                return jnp.where(lane == s16, tot, out_vec)

            out_vec = lax.fori_loop(0, L, s_body, jnp.zeros((L,), jnp.float32))
            out_v[pl.ds(g * L, L)] = out_vec + bias_vec

        pltpu.sync_copy(out_v, out_hbm.at[pl.ds(base + j * SUB, SUB)])

    def pair_body(t, _):
        for b in (0, 1):
            j = 2 * t + b
            nxt = j + 1

            @pl.when(nxt < NSUB)
            def _():
                gather_rows(nxt, 1 - b).start()

            gather_rows(j, b).wait()
            compute_sub(j, b)
        return 0

    lax.fori_loop(0, NSUB // 2, pair_body, 0)


def kernel(x, linear_w, v_w, b):
    xf = x.reshape(-1).astype(jnp.int32)
    lwf = linear_w.reshape(-1)
    b16 = jnp.broadcast_to(b.astype(jnp.float32), (L,))
    out = _fm_sc(xf, lwf, v_w, b16)
    return out.reshape(B, 1)
